# baseline (speedup 1.0000x reference)
import jax
import jax.numpy as jnp
from jax import lax
from jax.experimental import pallas as pl
from jax.experimental.pallas import tpu as pltpu

N = 4096
D_IN = 64
D_OUT = 64
K = 2
BN = 128


def _body(x_ref, adj_ref, w_ref, out_ref, ht_ref):
    @pl.when(pl.program_id(0) == 0)
    def _():
        h = jnp.dot(x_ref[...], w_ref[...],
                    preferred_element_type=jnp.float32)
        ht_ref[...] = h.T.astype(jnp.bfloat16)

    a = (adj_ref[0] + adj_ref[1]).astype(jnp.bfloat16)
    part_t = lax.dot_general(ht_ref[...], a, (((1,), (1,)), ((), ())),
                             preferred_element_type=jnp.float32)
    out_ref[...] = jnp.maximum(part_t, 0.0)


@jax.jit
def kernel(input, adj_list, W):
    out_t = pl.pallas_call(
        _body,
        grid=(N // BN,),
        in_specs=[
            pl.BlockSpec((N, D_IN), lambda i: (0, 0)),
            pl.BlockSpec((K, BN, N), lambda i: (0, i, 0)),
            pl.BlockSpec((D_IN, D_OUT), lambda i: (0, 0)),
        ],
        out_specs=pl.BlockSpec((D_OUT, BN), lambda i: (0, i)),
        out_shape=jax.ShapeDtypeStruct((D_OUT, N), jnp.float32),
        scratch_shapes=[pltpu.VMEM((D_OUT, N), jnp.bfloat16)],
    )(input, adj_list, W)
    return out_t.T


# speedup vs baseline: 1.0195x; 1.0195x over previous
import jax
import jax.numpy as jnp
from jax import lax
from jax.experimental import pallas as pl
from jax.experimental.pallas import tpu as pltpu

N = 4096
D_IN = 64
D_OUT = 64
K = 2
BN = 256
NB = N // BN


def _body(x_ref, adj_ref, w_ref, out_ref, ht_ref, acc_ref):
    i = pl.program_id(0)

    @pl.when(i == 0)
    def _():
        h = jnp.dot(x_ref[...], w_ref[...],
                    preferred_element_type=jnp.float32)
        ht_ref[...] = h.T.astype(jnp.bfloat16)

    a = (adj_ref[0] + adj_ref[1]).astype(jnp.bfloat16)
    part_t = lax.dot_general(ht_ref[...], a, (((1,), (1,)), ((), ())),
                             preferred_element_type=jnp.float32)
    acc_ref[:, pl.ds(i * BN, BN)] = jnp.maximum(part_t, 0.0)

    @pl.when(i == NB - 1)
    def _():
        out_ref[...] = acc_ref[...].T


@jax.jit
def kernel(input, adj_list, W):
    return pl.pallas_call(
        _body,
        grid=(NB,),
        in_specs=[
            pl.BlockSpec((N, D_IN), lambda i: (0, 0)),
            pl.BlockSpec((K, BN, N), lambda i: (0, i, 0)),
            pl.BlockSpec((D_IN, D_OUT), lambda i: (0, 0)),
        ],
        out_specs=pl.BlockSpec((N, D_OUT), lambda i: (0, 0)),
        out_shape=jax.ShapeDtypeStruct((N, D_OUT), jnp.float32),
        scratch_shapes=[
            pltpu.VMEM((D_OUT, N), jnp.bfloat16),
            pltpu.VMEM((D_OUT, N), jnp.float32),
        ],
    )(input, adj_list, W)


# speedup vs baseline: 1.1185x; 1.0971x over previous
import jax
import jax.numpy as jnp
from jax import lax
from jax.experimental import pallas as pl
from jax.experimental.pallas import tpu as pltpu

N = 4096
D_IN = 64
D_OUT = 64
K = 2
BN = 256


def _body(x_ref, adj_ref, w_ref, out_ref, ht_ref):
    @pl.when(pl.program_id(0) == 0)
    def _():
        ht = lax.dot_general(w_ref[...], x_ref[...], (((0,), (1,)), ((), ())),
                             preferred_element_type=jnp.float32)
        ht_ref[...] = ht.astype(jnp.bfloat16)

    a = (adj_ref[0] + adj_ref[1]).astype(jnp.bfloat16)
    part_t = lax.dot_general(ht_ref[...], a, (((1,), (1,)), ((), ())),
                             preferred_element_type=jnp.float32)
    out_ref[...] = jnp.maximum(part_t, 0.0)


@jax.jit
def kernel(input, adj_list, W):
    out_t = pl.pallas_call(
        _body,
        grid=(N // BN,),
        in_specs=[
            pl.BlockSpec((N, D_IN), lambda i: (0, 0)),
            pl.BlockSpec((K, BN, N), lambda i: (0, i, 0)),
            pl.BlockSpec((D_IN, D_OUT), lambda i: (0, 0)),
        ],
        out_specs=pl.BlockSpec((D_OUT, BN), lambda i: (0, i)),
        out_shape=jax.ShapeDtypeStruct((D_OUT, N), jnp.float32),
        scratch_shapes=[pltpu.VMEM((D_OUT, N), jnp.bfloat16)],
    )(input, adj_list, W)
    return out_t.T
